# baseline (speedup 1.0000x reference)
import functools

import jax
import jax.numpy as jnp
from jax import lax
from jax.experimental import pallas as pl
from jax.experimental.pallas import tpu as pltpu
from jax.experimental.pallas import tpu_sc as plsc

N = 10000
D = 128
NC = 2
NS = 16
NW = NC * NS
GB = 128
NP = 10240
ROWS_PER_TILE = NP // NS

_mesh = plsc.VectorSubcoreMesh(core_axis_name="c", subcore_axis_name="s")


def _make_spmm(gpt: int):
    @functools.partial(
        pl.kernel,
        mesh=_mesh,
        out_type=jax.ShapeDtypeStruct((NC, NP, D), jnp.float32),
        scratch_types=[
            [pltpu.VMEM((GB,), jnp.int32) for _ in range(2)],
            [pltpu.VMEM((GB,), jnp.int32) for _ in range(2)],
            [pltpu.VMEM((GB,), jnp.float32) for _ in range(2)],
            [pltpu.VMEM((GB,), jnp.int32) for _ in range(2)],
            [pltpu.VMEM((GB, D), jnp.float32) for _ in range(2)],
            pltpu.VMEM_SHARED((NP, D), jnp.float32),
            [pltpu.SemaphoreType.DMA for _ in range(2)],
            [pltpu.SemaphoreType.DMA for _ in range(2)],
            [pltpu.SemaphoreType.DMA for _ in range(2)],
        ],
    )
    def spmm(row_hbm, col_hbm, val_hbm, prev_hbm, zeros_hbm, out_hbm,
             rowb, colb, valb, rowsc, rbuf, acc, semg, sems, semi):
        cid = lax.axis_index("c")
        sid = lax.axis_index("s")
        wid = sid * NC + cid
        r0 = sid * ROWS_PER_TILE
        ebase = wid * (gpt * GB)

        def idx_load(g, b):
            off = ebase + g * GB
            pltpu.async_copy(row_hbm.at[pl.ds(off, GB)], rowb[b], semi[b])
            pltpu.async_copy(col_hbm.at[pl.ds(off, GB)], colb[b], semi[b])
            pltpu.async_copy(val_hbm.at[pl.ds(off, GB)], valb[b], semi[b])

        def idx_wait(b):
            pltpu.make_async_copy(row_hbm.at[pl.ds(0, GB)], rowb[b],
                                  semi[b]).wait()
            pltpu.make_async_copy(col_hbm.at[pl.ds(0, GB)], colb[b],
                                  semi[b]).wait()
            pltpu.make_async_copy(val_hbm.at[pl.ds(0, GB)], valb[b],
                                  semi[b]).wait()

        def gather_start(b):
            pltpu.async_copy(prev_hbm.at[colb[b]], rbuf[b], semg[b])

        def gather_wait(b):
            pltpu.make_async_copy(prev_hbm.at[colb[b]], rbuf[b],
                                  semg[b]).wait()

        def scatter_start(b):
            for j in range(GB // 16):
                rowsc[b][pl.ds(j * 16, 16)] = rowb[b][pl.ds(j * 16, 16)]
            pltpu.async_copy(rbuf[b], acc.at[rowsc[b]], sems[b], add=True)

        def scatter_wait(b):
            pltpu.make_async_copy(rbuf[b], acc.at[rowsc[b]], sems[b]).wait()

        def scale(b):
            def scale_body(k, c2):
                vv = valb[b][pl.ds(k * 16, 16)]
                for l in range(16):
                    sv = jnp.full((16,), vv[l], jnp.float32)
                    e = k * 16 + l
                    for j in range(D // 16):
                        rbuf[b][e, pl.ds(j * 16, 16)] = (
                            rbuf[b][e, pl.ds(j * 16, 16)] * sv)
                return c2

            lax.fori_loop(0, GB // 16, scale_body, 0, unroll=False)

        pltpu.sync_copy(zeros_hbm.at[pl.ds(r0, ROWS_PER_TILE)],
                        acc.at[pl.ds(r0, ROWS_PER_TILE)])
        plsc.subcore_barrier()

        idx_load(0, 0)
        idx_load(1, 1)
        idx_wait(0)
        gather_start(0)

        def outer(t, carry):
            g0 = 2 * t
            g1 = g0 + 1
            gather_wait(0)

            @pl.when(t > 0)
            def _():
                scatter_wait(1)
            idx_wait(1)
            gather_start(1)
            scale(0)
            scatter_start(0)

            @pl.when(g0 + 2 < gpt)
            def _():
                idx_load(g0 + 2, 0)
            gather_wait(1)
            scatter_wait(0)

            @pl.when(g1 + 1 < gpt)
            def _():
                idx_wait(0)
                gather_start(0)
            scale(1)
            scatter_start(1)

            @pl.when(g1 + 2 < gpt)
            def _():
                idx_load(g1 + 2, 1)
            return carry

        lax.fori_loop(0, gpt // 2, outer, 0, unroll=False)
        scatter_wait(1)
        plsc.subcore_barrier()

        pltpu.sync_copy(acc.at[pl.ds(r0, ROWS_PER_TILE)],
                        out_hbm.at[cid, pl.ds(r0, ROWS_PER_TILE)])

    return spmm


_ROW_BLK = 1000


def _dense_body(a0_ref, a1_ref, prev_ref, w1_ref, w2_ref, b1_ref, b2_ref,
                out_ref):
    a = a0_ref[...] + a1_ref[...]
    x2 = prev_ref[...] * a
    out_ref[...] = (
        jnp.dot(a, w1_ref[...], preferred_element_type=jnp.float32)
        + jnp.dot(x2, w2_ref[...], preferred_element_type=jnp.float32)
        + b1_ref[...] + b2_ref[...]
    )


def _dense(a0, a1, prev, w1, w2, b1, b2):
    grid = (N // _ROW_BLK,)
    row_spec = pl.BlockSpec((_ROW_BLK, D), lambda i: (i, 0))
    full_spec = pl.BlockSpec((D, D), lambda i: (0, 0))
    bias_spec = pl.BlockSpec((1, D), lambda i: (0, 0))
    return pl.pallas_call(
        _dense_body,
        grid=grid,
        in_specs=[row_spec, row_spec, row_spec, full_spec, full_spec,
                  bias_spec, bias_spec],
        out_specs=row_spec,
        out_shape=jax.ShapeDtypeStruct((N, D), jnp.float32),
    )(a0, a1, prev, w1, w2, b1, b2)


def kernel(L_I_indices, L_I_values, prev_embeddings, W1, W2, b1, b2):
    e = L_I_values.shape[0]
    gpt = -(-e // (NW * GB))
    gpt += gpt % 2
    e_pad = gpt * NW * GB
    pad = e_pad - e

    row = L_I_indices[0]
    col = L_I_indices[1]
    if pad:
        zi = jnp.zeros((pad,), jnp.int32)
        row = jnp.concatenate([row, zi])
        col = jnp.concatenate([col, zi])
        vals = jnp.concatenate([L_I_values, jnp.zeros((pad,), jnp.float32)])
    else:
        vals = L_I_values

    zeros = jnp.zeros((NP, D), jnp.float32)
    partial = _make_spmm(gpt)(row, col, vals, prev_embeddings, zeros)
    return _dense(partial[0, :N], partial[1, :N], prev_embeddings, W1, W2,
                  b1, b2)


# speedup vs baseline: 1.0364x; 1.0364x over previous
import functools

import jax
import jax.numpy as jnp
from jax import lax
from jax.experimental import pallas as pl
from jax.experimental.pallas import tpu as pltpu
from jax.experimental.pallas import tpu_sc as plsc

N = 10000
D = 128
NC = 2
NS = 16
NW = NC * NS
GB = 64
NB = 4
NP = 10240
ROWS_PER_TILE = NP // NS

_mesh = plsc.VectorSubcoreMesh(core_axis_name="c", subcore_axis_name="s")


def _make_spmm(gpt: int):
    @functools.partial(
        pl.kernel,
        mesh=_mesh,
        out_type=jax.ShapeDtypeStruct((NC, NP, D), jnp.float32),
        scratch_types=[
            [pltpu.VMEM((GB,), jnp.int32) for _ in range(NB)],
            [pltpu.VMEM((GB,), jnp.int32) for _ in range(NB)],
            [pltpu.VMEM((GB,), jnp.float32) for _ in range(NB)],
            [pltpu.VMEM((GB,), jnp.int32) for _ in range(NB)],
            [pltpu.VMEM((GB, D), jnp.float32) for _ in range(NB)],
            pltpu.VMEM_SHARED((NP, D), jnp.float32),
            [pltpu.SemaphoreType.DMA for _ in range(NB)],
            [pltpu.SemaphoreType.DMA for _ in range(NB)],
            [pltpu.SemaphoreType.DMA for _ in range(NB)],
        ],
    )
    def spmm(row_hbm, col_hbm, val_hbm, prev_hbm, zeros_hbm, out_hbm,
             rowb, colb, valb, rowsc, rbuf, acc, semg, sems, semi):
        cid = lax.axis_index("c")
        sid = lax.axis_index("s")
        wid = sid * NC + cid
        r0 = sid * ROWS_PER_TILE
        ebase = wid * (gpt * GB)

        def idx_load(g, b):
            off = ebase + g * GB
            pltpu.async_copy(row_hbm.at[pl.ds(off, GB)], rowb[b], semi[b])
            pltpu.async_copy(col_hbm.at[pl.ds(off, GB)], colb[b], semi[b])
            pltpu.async_copy(val_hbm.at[pl.ds(off, GB)], valb[b], semi[b])

        def idx_wait(b):
            pltpu.make_async_copy(row_hbm.at[pl.ds(0, GB)], rowb[b],
                                  semi[b]).wait()
            pltpu.make_async_copy(col_hbm.at[pl.ds(0, GB)], colb[b],
                                  semi[b]).wait()
            pltpu.make_async_copy(val_hbm.at[pl.ds(0, GB)], valb[b],
                                  semi[b]).wait()

        def gather_start(b):
            pltpu.async_copy(prev_hbm.at[colb[b]], rbuf[b], semg[b])

        def gather_wait(b):
            pltpu.make_async_copy(prev_hbm.at[colb[b]], rbuf[b],
                                  semg[b]).wait()

        def scatter_start(b):
            for j in range(GB // 16):
                rowsc[b][pl.ds(j * 16, 16)] = rowb[b][pl.ds(j * 16, 16)]
            pltpu.async_copy(rbuf[b], acc.at[rowsc[b]], sems[b], add=True)

        def scatter_wait(b):
            pltpu.make_async_copy(rbuf[b], acc.at[rowsc[b]], sems[b]).wait()

        def scale(b):
            def scale_body(k, c2):
                vv = valb[b][pl.ds(k * 16, 16)]
                for l in range(16):
                    sv = jnp.full((16,), vv[l], jnp.float32)
                    e = k * 16 + l
                    for j in range(D // 16):
                        rbuf[b][e, pl.ds(j * 16, 16)] = (
                            rbuf[b][e, pl.ds(j * 16, 16)] * sv)
                return c2

            lax.fori_loop(0, GB // 16, scale_body, 0, unroll=False)

        pltpu.sync_copy(zeros_hbm.at[pl.ds(r0, ROWS_PER_TILE)],
                        acc.at[pl.ds(r0, ROWS_PER_TILE)])
        plsc.subcore_barrier()

        for b in range(NB - 1):
            idx_load(b, b)
        for b in range(NB - 1):
            idx_wait(b)
            gather_start(b)
        idx_load(NB - 1, NB - 1)

        def outer(t, carry):
            for b in range(NB):
                g = NB * t + b
                bn = (b + NB - 1) % NB
                gather_wait(b)

                @pl.when(g > 0)
                def _():
                    scatter_wait(bn)

                @pl.when(g + NB - 1 < gpt)
                def _():
                    idx_wait(bn)
                    gather_start(bn)
                scale(b)
                scatter_start(b)

                @pl.when(g + NB < gpt)
                def _():
                    idx_load(g + NB, b)
            return carry

        lax.fori_loop(0, gpt // NB, outer, 0, unroll=False)
        scatter_wait(NB - 1)
        plsc.subcore_barrier()

        pltpu.sync_copy(acc.at[pl.ds(r0, ROWS_PER_TILE)],
                        out_hbm.at[cid, pl.ds(r0, ROWS_PER_TILE)])

    return spmm


_ROW_BLK = 1000


def _dense_body(a0_ref, a1_ref, prev_ref, w1_ref, w2_ref, b1_ref, b2_ref,
                out_ref):
    a = a0_ref[...] + a1_ref[...]
    x2 = prev_ref[...] * a
    out_ref[...] = (
        jnp.dot(a, w1_ref[...], preferred_element_type=jnp.float32)
        + jnp.dot(x2, w2_ref[...], preferred_element_type=jnp.float32)
        + b1_ref[...] + b2_ref[...]
    )


def _dense(a0, a1, prev, w1, w2, b1, b2):
    grid = (N // _ROW_BLK,)
    row_spec = pl.BlockSpec((_ROW_BLK, D), lambda i: (i, 0))
    full_spec = pl.BlockSpec((D, D), lambda i: (0, 0))
    bias_spec = pl.BlockSpec((1, D), lambda i: (0, 0))
    return pl.pallas_call(
        _dense_body,
        grid=grid,
        in_specs=[row_spec, row_spec, row_spec, full_spec, full_spec,
                  bias_spec, bias_spec],
        out_specs=row_spec,
        out_shape=jax.ShapeDtypeStruct((N, D), jnp.float32),
    )(a0, a1, prev, w1, w2, b1, b2)


def kernel(L_I_indices, L_I_values, prev_embeddings, W1, W2, b1, b2):
    e = L_I_values.shape[0]
    gpt = -(-e // (NW * GB))
    gpt = -(-gpt // NB) * NB
    e_pad = gpt * NW * GB
    pad = e_pad - e

    row = L_I_indices[0]
    col = L_I_indices[1]
    if pad:
        zi = jnp.zeros((pad,), jnp.int32)
        row = jnp.concatenate([row, zi])
        col = jnp.concatenate([col, zi])
        vals = jnp.concatenate([L_I_values, jnp.zeros((pad,), jnp.float32)])
    else:
        vals = L_I_values

    zeros = jnp.zeros((NP, D), jnp.float32)
    partial = _make_spmm(gpt)(row, col, vals, prev_embeddings, zeros)
    return _dense(partial[0, :N], partial[1, :N], prev_embeddings, W1, W2,
                  b1, b2)


# speedup vs baseline: 1.3262x; 1.2796x over previous
import functools

import jax
import jax.numpy as jnp
from jax import lax
from jax.experimental import pallas as pl
from jax.experimental.pallas import tpu as pltpu
from jax.experimental.pallas import tpu_sc as plsc

N = 10000
D = 128
NC = 2
NS = 16
NW = NC * NS
GB = 64
NB = 4
F0 = 0.25
NP = 10240
ROWS_PER_TILE = NP // NS

_mesh = plsc.VectorSubcoreMesh(core_axis_name="c", subcore_axis_name="s")


def _make_spmm(gpt0: int, gpt1: int):
    @functools.partial(
        pl.kernel,
        mesh=_mesh,
        out_type=jax.ShapeDtypeStruct((NC, NP, D), jnp.float32),
        scratch_types=[
            [pltpu.VMEM((GB,), jnp.int32) for _ in range(NB)],
            [pltpu.VMEM((GB,), jnp.int32) for _ in range(NB)],
            [pltpu.VMEM((GB,), jnp.float32) for _ in range(NB)],
            [pltpu.VMEM((GB,), jnp.int32) for _ in range(NB)],
            [pltpu.VMEM((GB, D), jnp.float32) for _ in range(NB)],
            pltpu.VMEM_SHARED((NP, D), jnp.float32),
            [pltpu.SemaphoreType.DMA for _ in range(NB)],
            [pltpu.SemaphoreType.DMA for _ in range(NB)],
            [pltpu.SemaphoreType.DMA for _ in range(NB)],
        ],
    )
    def spmm(row_hbm, col_hbm, val_hbm, prev_hbm, zeros_hbm, out_hbm,
             rowb, colb, valb, rowsc, rbuf, acc, semg, sems, semi):
        cid = lax.axis_index("c")
        sid = lax.axis_index("s")
        r0 = sid * ROWS_PER_TILE

        def idx_load(g, b, ebase):
            off = ebase + g * GB
            pltpu.async_copy(row_hbm.at[pl.ds(off, GB)], rowb[b], semi[b])
            pltpu.async_copy(col_hbm.at[pl.ds(off, GB)], colb[b], semi[b])
            pltpu.async_copy(val_hbm.at[pl.ds(off, GB)], valb[b], semi[b])

        def idx_wait(b):
            pltpu.make_async_copy(row_hbm.at[pl.ds(0, GB)], rowb[b],
                                  semi[b]).wait()
            pltpu.make_async_copy(col_hbm.at[pl.ds(0, GB)], colb[b],
                                  semi[b]).wait()
            pltpu.make_async_copy(val_hbm.at[pl.ds(0, GB)], valb[b],
                                  semi[b]).wait()

        def gather_start(b):
            pltpu.async_copy(prev_hbm.at[colb[b]], rbuf[b], semg[b])

        def gather_wait(b):
            pltpu.make_async_copy(prev_hbm.at[colb[b]], rbuf[b],
                                  semg[b]).wait()

        def scatter_start(b):
            for j in range(GB // 16):
                rowsc[b][pl.ds(j * 16, 16)] = rowb[b][pl.ds(j * 16, 16)]
            pltpu.async_copy(rbuf[b], acc.at[rowsc[b]], sems[b], add=True)

        def scatter_wait(b):
            pltpu.make_async_copy(rbuf[b], acc.at[rowsc[b]], sems[b]).wait()

        def scale(b):
            def scale_body(k, c2):
                vv = valb[b][pl.ds(k * 16, 16)]
                for l in range(16):
                    sv = jnp.full((16,), vv[l], jnp.float32)
                    e = k * 16 + l
                    for j in range(D // 16):
                        rbuf[b][e, pl.ds(j * 16, 16)] = (
                            rbuf[b][e, pl.ds(j * 16, 16)] * sv)
                return c2

            lax.fori_loop(0, GB // 16, scale_body, 0, unroll=False)

        pltpu.sync_copy(zeros_hbm.at[pl.ds(r0, ROWS_PER_TILE)],
                        acc.at[pl.ds(r0, ROWS_PER_TILE)])
        plsc.subcore_barrier()

        def run_core(gpt, ebase):
            for b in range(NB - 1):
                idx_load(b, b, ebase)
            for b in range(NB - 1):
                idx_wait(b)
                gather_start(b)
            idx_load(NB - 1, NB - 1, ebase)

            def outer(t, carry):
                for b in range(NB):
                    g = NB * t + b
                    bn = (b + NB - 1) % NB
                    gather_wait(b)

                    @pl.when(g > 0)
                    def _():
                        scatter_wait(bn)

                    @pl.when(g + NB - 1 < gpt)
                    def _():
                        idx_wait(bn)
                        gather_start(bn)
                    scale(b)
                    scatter_start(b)

                    @pl.when(g + NB < gpt)
                    def _():
                        idx_load(g + NB, b, ebase)
                return carry

            lax.fori_loop(0, gpt // NB, outer, 0, unroll=False)
            scatter_wait(NB - 1)

        @pl.when(cid == 0)
        def _():
            run_core(gpt0, sid * (gpt0 * GB))

        @pl.when(cid == 1)
        def _():
            run_core(gpt1, NS * (gpt0 * GB) + sid * (gpt1 * GB))

        plsc.subcore_barrier()

        pltpu.sync_copy(acc.at[pl.ds(r0, ROWS_PER_TILE)],
                        out_hbm.at[cid, pl.ds(r0, ROWS_PER_TILE)])

    return spmm


_ROW_BLK = 1000


def _dense_body(a0_ref, a1_ref, prev_ref, w1_ref, w2_ref, b1_ref, b2_ref,
                out_ref):
    a = a0_ref[...] + a1_ref[...]
    x2 = prev_ref[...] * a
    out_ref[...] = (
        jnp.dot(a, w1_ref[...], preferred_element_type=jnp.float32)
        + jnp.dot(x2, w2_ref[...], preferred_element_type=jnp.float32)
        + b1_ref[...] + b2_ref[...]
    )


def _dense(a0, a1, prev, w1, w2, b1, b2):
    grid = (N // _ROW_BLK,)
    row_spec = pl.BlockSpec((_ROW_BLK, D), lambda i: (i, 0))
    full_spec = pl.BlockSpec((D, D), lambda i: (0, 0))
    bias_spec = pl.BlockSpec((1, D), lambda i: (0, 0))
    return pl.pallas_call(
        _dense_body,
        grid=grid,
        in_specs=[row_spec, row_spec, row_spec, full_spec, full_spec,
                  bias_spec, bias_spec],
        out_specs=row_spec,
        out_shape=jax.ShapeDtypeStruct((N, D), jnp.float32),
    )(a0, a1, prev, w1, w2, b1, b2)


def kernel(L_I_indices, L_I_values, prev_embeddings, W1, W2, b1, b2):
    e = L_I_values.shape[0]
    gtot = -(-e // (NS * GB))
    gpt0 = max(NB, (int(gtot * F0) // NB) * NB)
    gpt1 = -(-(gtot - gpt0) // NB) * NB
    e_pad = (gpt0 + gpt1) * NS * GB
    pad = e_pad - e

    row = L_I_indices[0]
    col = L_I_indices[1]
    if pad:
        zi = jnp.zeros((pad,), jnp.int32)
        row = jnp.concatenate([row, zi])
        col = jnp.concatenate([col, zi])
        vals = jnp.concatenate([L_I_values, jnp.zeros((pad,), jnp.float32)])
    else:
        vals = L_I_values

    zeros = jnp.zeros((NP, D), jnp.float32)
    partial = _make_spmm(gpt0, gpt1)(row, col, vals, prev_embeddings, zeros)
    return _dense(partial[0, :N], partial[1, :N], prev_embeddings, W1, W2,
                  b1, b2)


# speedup vs baseline: 1.4159x; 1.0676x over previous
import functools

import jax
import jax.numpy as jnp
from jax import lax
from jax.experimental import pallas as pl
from jax.experimental.pallas import tpu as pltpu
from jax.experimental.pallas import tpu_sc as plsc

N = 10000
D = 128
NC = 2
NS = 16
NW = NC * NS
GB = 64
NB = 4
F0 = 0.35
NP = 10240
ROWS_PER_TILE = NP // NS

_mesh = plsc.VectorSubcoreMesh(core_axis_name="c", subcore_axis_name="s")


def _make_spmm(gpt0: int, gpt1: int):
    @functools.partial(
        pl.kernel,
        mesh=_mesh,
        out_type=jax.ShapeDtypeStruct((NC, NP, D), jnp.float32),
        scratch_types=[
            [pltpu.VMEM((GB,), jnp.int32) for _ in range(NB)],
            [pltpu.VMEM((GB,), jnp.int32) for _ in range(NB)],
            [pltpu.VMEM((GB,), jnp.float32) for _ in range(NB)],
            [pltpu.VMEM((GB,), jnp.int32) for _ in range(NB)],
            [pltpu.VMEM((GB, D), jnp.float32) for _ in range(NB)],
            pltpu.VMEM_SHARED((NP, D), jnp.float32),
            [pltpu.SemaphoreType.DMA for _ in range(NB)],
            [pltpu.SemaphoreType.DMA for _ in range(NB)],
            [pltpu.SemaphoreType.DMA for _ in range(NB)],
        ],
    )
    def spmm(row_hbm, col_hbm, val_hbm, prev_hbm, zeros_hbm, out_hbm,
             rowb, colb, valb, rowsc, rbuf, acc, semg, sems, semi):
        cid = lax.axis_index("c")
        sid = lax.axis_index("s")
        r0 = sid * ROWS_PER_TILE

        def idx_load(g, b, ebase):
            off = ebase + g * GB
            pltpu.async_copy(row_hbm.at[pl.ds(off, GB)], rowb[b], semi[b])
            pltpu.async_copy(col_hbm.at[pl.ds(off, GB)], colb[b], semi[b])
            pltpu.async_copy(val_hbm.at[pl.ds(off, GB)], valb[b], semi[b])

        def idx_wait(b):
            pltpu.make_async_copy(row_hbm.at[pl.ds(0, GB)], rowb[b],
                                  semi[b]).wait()
            pltpu.make_async_copy(col_hbm.at[pl.ds(0, GB)], colb[b],
                                  semi[b]).wait()
            pltpu.make_async_copy(val_hbm.at[pl.ds(0, GB)], valb[b],
                                  semi[b]).wait()

        def gather_start(b):
            pltpu.async_copy(prev_hbm.at[colb[b]], rbuf[b], semg[b])

        def gather_wait(b):
            pltpu.make_async_copy(prev_hbm.at[colb[b]], rbuf[b],
                                  semg[b]).wait()

        def scatter_start(b):
            for j in range(GB // 16):
                rowsc[b][pl.ds(j * 16, 16)] = rowb[b][pl.ds(j * 16, 16)]
            pltpu.async_copy(rbuf[b], acc.at[rowsc[b]], sems[b], add=True)

        def scatter_wait(b):
            pltpu.make_async_copy(rbuf[b], acc.at[rowsc[b]], sems[b]).wait()

        def scale(b):
            def scale_body(k, c2):
                vv = valb[b][pl.ds(k * 16, 16)]
                for l in range(16):
                    sv = jnp.full((16,), vv[l], jnp.float32)
                    e = k * 16 + l
                    for j in range(D // 16):
                        rbuf[b][e, pl.ds(j * 16, 16)] = (
                            rbuf[b][e, pl.ds(j * 16, 16)] * sv)
                return c2

            lax.fori_loop(0, GB // 16, scale_body, 0, unroll=False)

        pltpu.sync_copy(zeros_hbm.at[pl.ds(r0, ROWS_PER_TILE)],
                        acc.at[pl.ds(r0, ROWS_PER_TILE)])
        plsc.subcore_barrier()

        def run_core(gpt, ebase):
            for b in range(NB - 1):
                idx_load(b, b, ebase)
            for b in range(NB - 1):
                idx_wait(b)
                gather_start(b)
            idx_load(NB - 1, NB - 1, ebase)

            def outer(t, carry):
                for b in range(NB):
                    g = NB * t + b
                    bn = (b + NB - 1) % NB
                    gather_wait(b)

                    @pl.when(g > 0)
                    def _():
                        scatter_wait(bn)

                    @pl.when(g + NB - 1 < gpt)
                    def _():
                        idx_wait(bn)
                        gather_start(bn)
                    scale(b)
                    scatter_start(b)

                    @pl.when(g + NB < gpt)
                    def _():
                        idx_load(g + NB, b, ebase)
                return carry

            lax.fori_loop(0, gpt // NB, outer, 0, unroll=False)
            scatter_wait(NB - 1)

        @pl.when(cid == 0)
        def _():
            run_core(gpt0, sid * (gpt0 * GB))

        @pl.when(cid == 1)
        def _():
            run_core(gpt1, NS * (gpt0 * GB) + sid * (gpt1 * GB))

        plsc.subcore_barrier()

        pltpu.sync_copy(acc.at[pl.ds(r0, ROWS_PER_TILE)],
                        out_hbm.at[cid, pl.ds(r0, ROWS_PER_TILE)])

    return spmm


_ROW_BLK = 1000


def _dense_body(a0_ref, a1_ref, prev_ref, w1_ref, w2_ref, b1_ref, b2_ref,
                out_ref):
    a = a0_ref[...] + a1_ref[...]
    x2 = prev_ref[...] * a
    out_ref[...] = (
        jnp.dot(a, w1_ref[...], preferred_element_type=jnp.float32)
        + jnp.dot(x2, w2_ref[...], preferred_element_type=jnp.float32)
        + b1_ref[...] + b2_ref[...]
    )


def _dense(a0, a1, prev, w1, w2, b1, b2):
    grid = (N // _ROW_BLK,)
    row_spec = pl.BlockSpec((_ROW_BLK, D), lambda i: (i, 0))
    full_spec = pl.BlockSpec((D, D), lambda i: (0, 0))
    bias_spec = pl.BlockSpec((1, D), lambda i: (0, 0))
    return pl.pallas_call(
        _dense_body,
        grid=grid,
        in_specs=[row_spec, row_spec, row_spec, full_spec, full_spec,
                  bias_spec, bias_spec],
        out_specs=row_spec,
        out_shape=jax.ShapeDtypeStruct((N, D), jnp.float32),
    )(a0, a1, prev, w1, w2, b1, b2)


def kernel(L_I_indices, L_I_values, prev_embeddings, W1, W2, b1, b2):
    e = L_I_values.shape[0]
    gtot = -(-e // (NS * GB))
    gpt0 = max(NB, (int(gtot * F0) // NB) * NB)
    gpt1 = -(-(gtot - gpt0) // NB) * NB
    e_pad = (gpt0 + gpt1) * NS * GB
    pad = e_pad - e

    row = L_I_indices[0]
    col = L_I_indices[1]
    if pad:
        zi = jnp.zeros((pad,), jnp.int32)
        row = jnp.concatenate([row, zi])
        col = jnp.concatenate([col, zi])
        vals = jnp.concatenate([L_I_values, jnp.zeros((pad,), jnp.float32)])
    else:
        vals = L_I_values

    zeros = jnp.zeros((NP, D), jnp.float32)
    partial = _make_spmm(gpt0, gpt1)(row, col, vals, prev_embeddings, zeros)
    return _dense(partial[0, :N], partial[1, :N], prev_embeddings, W1, W2,
                  b1, b2)


# speedup vs baseline: 1.4663x; 1.0356x over previous
import functools

import jax
import jax.numpy as jnp
from jax import lax
from jax.experimental import pallas as pl
from jax.experimental.pallas import tpu as pltpu
from jax.experimental.pallas import tpu_sc as plsc

N = 10000
D = 128
NC = 2
NS = 16
NW = NC * NS
GB = 64
NB = 4
F0 = 0.42
NP = 10240
ROWS_PER_TILE = NP // NS

_mesh = plsc.VectorSubcoreMesh(core_axis_name="c", subcore_axis_name="s")


def _make_spmm(gpt0: int, gpt1: int):
    @functools.partial(
        pl.kernel,
        mesh=_mesh,
        out_type=jax.ShapeDtypeStruct((NC, NP, D), jnp.float32),
        scratch_types=[
            [pltpu.VMEM((GB,), jnp.int32) for _ in range(NB)],
            [pltpu.VMEM((GB,), jnp.int32) for _ in range(NB)],
            [pltpu.VMEM((GB,), jnp.float32) for _ in range(NB)],
            [pltpu.VMEM((GB,), jnp.int32) for _ in range(NB)],
            [pltpu.VMEM((GB, D), jnp.float32) for _ in range(NB)],
            pltpu.VMEM_SHARED((NP, D), jnp.float32),
            [pltpu.SemaphoreType.DMA for _ in range(NB)],
            [pltpu.SemaphoreType.DMA for _ in range(NB)],
            [pltpu.SemaphoreType.DMA for _ in range(NB)],
        ],
    )
    def spmm(row_hbm, col_hbm, val_hbm, prev_hbm, zeros_hbm, out_hbm,
             rowb, colb, valb, rowsc, rbuf, acc, semg, sems, semi):
        cid = lax.axis_index("c")
        sid = lax.axis_index("s")
        r0 = sid * ROWS_PER_TILE

        def idx_load(g, b, ebase):
            off = ebase + g * GB
            pltpu.async_copy(row_hbm.at[pl.ds(off, GB)], rowb[b], semi[b])
            pltpu.async_copy(col_hbm.at[pl.ds(off, GB)], colb[b], semi[b])
            pltpu.async_copy(val_hbm.at[pl.ds(off, GB)], valb[b], semi[b])

        def idx_wait(b):
            pltpu.make_async_copy(row_hbm.at[pl.ds(0, GB)], rowb[b],
                                  semi[b]).wait()
            pltpu.make_async_copy(col_hbm.at[pl.ds(0, GB)], colb[b],
                                  semi[b]).wait()
            pltpu.make_async_copy(val_hbm.at[pl.ds(0, GB)], valb[b],
                                  semi[b]).wait()

        def gather_start(b):
            pltpu.async_copy(prev_hbm.at[colb[b]], rbuf[b], semg[b])

        def gather_wait(b):
            pltpu.make_async_copy(prev_hbm.at[colb[b]], rbuf[b],
                                  semg[b]).wait()

        def scatter_start(b):
            for j in range(GB // 16):
                rowsc[b][pl.ds(j * 16, 16)] = rowb[b][pl.ds(j * 16, 16)]
            pltpu.async_copy(rbuf[b], acc.at[rowsc[b]], sems[b], add=True)

        def scatter_wait(b):
            pltpu.make_async_copy(rbuf[b], acc.at[rowsc[b]], sems[b]).wait()

        def scale(b):
            def scale_body(k, c2):
                vv = valb[b][pl.ds(k * 16, 16)]
                for l in range(16):
                    sv = jnp.full((16,), vv[l], jnp.float32)
                    e = k * 16 + l
                    for j in range(D // 16):
                        rbuf[b][e, pl.ds(j * 16, 16)] = (
                            rbuf[b][e, pl.ds(j * 16, 16)] * sv)
                return c2

            lax.fori_loop(0, GB // 16, scale_body, 0, unroll=False)

        pltpu.sync_copy(zeros_hbm.at[pl.ds(r0, ROWS_PER_TILE)],
                        acc.at[pl.ds(r0, ROWS_PER_TILE)])
        plsc.subcore_barrier()

        def run_core(gpt, ebase):
            for b in range(NB - 1):
                idx_load(b, b, ebase)
            for b in range(NB - 1):
                idx_wait(b)
                gather_start(b)
            idx_load(NB - 1, NB - 1, ebase)

            def outer(t, carry):
                for b in range(NB):
                    g = NB * t + b
                    bn = (b + NB - 1) % NB
                    gather_wait(b)

                    @pl.when(g > 0)
                    def _():
                        scatter_wait(bn)

                    @pl.when(g + NB - 1 < gpt)
                    def _():
                        idx_wait(bn)
                        gather_start(bn)
                    scale(b)
                    scatter_start(b)

                    @pl.when(g + NB < gpt)
                    def _():
                        idx_load(g + NB, b, ebase)
                return carry

            lax.fori_loop(0, gpt // NB, outer, 0, unroll=False)
            scatter_wait(NB - 1)

        @pl.when(cid == 0)
        def _():
            run_core(gpt0, sid * (gpt0 * GB))

        @pl.when(cid == 1)
        def _():
            run_core(gpt1, NS * (gpt0 * GB) + sid * (gpt1 * GB))

        plsc.subcore_barrier()

        pltpu.sync_copy(acc.at[pl.ds(r0, ROWS_PER_TILE)],
                        out_hbm.at[cid, pl.ds(r0, ROWS_PER_TILE)])

    return spmm


_ROW_BLK = 1000


def _dense_body(a0_ref, a1_ref, prev_ref, w1_ref, w2_ref, b1_ref, b2_ref,
                out_ref):
    a = a0_ref[...] + a1_ref[...]
    x2 = prev_ref[...] * a
    out_ref[...] = (
        jnp.dot(a, w1_ref[...], preferred_element_type=jnp.float32)
        + jnp.dot(x2, w2_ref[...], preferred_element_type=jnp.float32)
        + b1_ref[...] + b2_ref[...]
    )


def _dense(a0, a1, prev, w1, w2, b1, b2):
    grid = (N // _ROW_BLK,)
    row_spec = pl.BlockSpec((_ROW_BLK, D), lambda i: (i, 0))
    full_spec = pl.BlockSpec((D, D), lambda i: (0, 0))
    bias_spec = pl.BlockSpec((1, D), lambda i: (0, 0))
    return pl.pallas_call(
        _dense_body,
        grid=grid,
        in_specs=[row_spec, row_spec, row_spec, full_spec, full_spec,
                  bias_spec, bias_spec],
        out_specs=row_spec,
        out_shape=jax.ShapeDtypeStruct((N, D), jnp.float32),
    )(a0, a1, prev, w1, w2, b1, b2)


def kernel(L_I_indices, L_I_values, prev_embeddings, W1, W2, b1, b2):
    e = L_I_values.shape[0]
    gtot = -(-e // (NS * GB))
    gpt0 = max(NB, (int(gtot * F0) // NB) * NB)
    gpt1 = -(-(gtot - gpt0) // NB) * NB
    e_pad = (gpt0 + gpt1) * NS * GB
    pad = e_pad - e

    row = L_I_indices[0]
    col = L_I_indices[1]
    if pad:
        zi = jnp.zeros((pad,), jnp.int32)
        row = jnp.concatenate([row, zi])
        col = jnp.concatenate([col, zi])
        vals = jnp.concatenate([L_I_values, jnp.zeros((pad,), jnp.float32)])
    else:
        vals = L_I_values

    zeros = jnp.zeros((NP, D), jnp.float32)
    partial = _make_spmm(gpt0, gpt1)(row, col, vals, prev_embeddings, zeros)
    return _dense(partial[0, :N], partial[1, :N], prev_embeddings, W1, W2,
                  b1, b2)


# speedup vs baseline: 1.5173x; 1.0348x over previous
import functools

import jax
import jax.numpy as jnp
from jax import lax
from jax.experimental import pallas as pl
from jax.experimental.pallas import tpu as pltpu
from jax.experimental.pallas import tpu_sc as plsc

N = 10000
D = 128
NC = 2
NS = 16
NW = NC * NS
GB = 64
NB = 4
F0 = 0.48
NP = 10240
ROWS_PER_TILE = NP // NS

_mesh = plsc.VectorSubcoreMesh(core_axis_name="c", subcore_axis_name="s")


def _make_spmm(gpt0: int, gpt1: int):
    @functools.partial(
        pl.kernel,
        mesh=_mesh,
        out_type=jax.ShapeDtypeStruct((NC, NP, D), jnp.float32),
        scratch_types=[
            [pltpu.VMEM((GB,), jnp.int32) for _ in range(NB)],
            [pltpu.VMEM((GB,), jnp.int32) for _ in range(NB)],
            [pltpu.VMEM((GB,), jnp.float32) for _ in range(NB)],
            [pltpu.VMEM((GB,), jnp.int32) for _ in range(NB)],
            [pltpu.VMEM((GB, D), jnp.float32) for _ in range(NB)],
            pltpu.VMEM_SHARED((NP, D), jnp.float32),
            [pltpu.SemaphoreType.DMA for _ in range(NB)],
            [pltpu.SemaphoreType.DMA for _ in range(NB)],
            [pltpu.SemaphoreType.DMA for _ in range(NB)],
        ],
    )
    def spmm(row_hbm, col_hbm, val_hbm, prev_hbm, zeros_hbm, out_hbm,
             rowb, colb, valb, rowsc, rbuf, acc, semg, sems, semi):
        cid = lax.axis_index("c")
        sid = lax.axis_index("s")
        r0 = sid * ROWS_PER_TILE

        def idx_load(g, b, ebase):
            off = ebase + g * GB
            pltpu.async_copy(row_hbm.at[pl.ds(off, GB)], rowb[b], semi[b])
            pltpu.async_copy(col_hbm.at[pl.ds(off, GB)], colb[b], semi[b])
            pltpu.async_copy(val_hbm.at[pl.ds(off, GB)], valb[b], semi[b])

        def idx_wait(b):
            pltpu.make_async_copy(row_hbm.at[pl.ds(0, GB)], rowb[b],
                                  semi[b]).wait()
            pltpu.make_async_copy(col_hbm.at[pl.ds(0, GB)], colb[b],
                                  semi[b]).wait()
            pltpu.make_async_copy(val_hbm.at[pl.ds(0, GB)], valb[b],
                                  semi[b]).wait()

        def gather_start(b):
            pltpu.async_copy(prev_hbm.at[colb[b]], rbuf[b], semg[b])

        def gather_wait(b):
            pltpu.make_async_copy(prev_hbm.at[colb[b]], rbuf[b],
                                  semg[b]).wait()

        def scatter_start(b):
            for j in range(GB // 16):
                rowsc[b][pl.ds(j * 16, 16)] = rowb[b][pl.ds(j * 16, 16)]
            pltpu.async_copy(rbuf[b], acc.at[rowsc[b]], sems[b], add=True)

        def scatter_wait(b):
            pltpu.make_async_copy(rbuf[b], acc.at[rowsc[b]], sems[b]).wait()

        def scale(b):
            def scale_body(k, c2):
                vv = valb[b][pl.ds(k * 16, 16)]
                for l in range(16):
                    sv = jnp.full((16,), vv[l], jnp.float32)
                    e = k * 16 + l
                    for j in range(D // 16):
                        rbuf[b][e, pl.ds(j * 16, 16)] = (
                            rbuf[b][e, pl.ds(j * 16, 16)] * sv)
                return c2

            lax.fori_loop(0, GB // 16, scale_body, 0, unroll=False)

        pltpu.sync_copy(zeros_hbm.at[pl.ds(r0, ROWS_PER_TILE)],
                        acc.at[pl.ds(r0, ROWS_PER_TILE)])
        plsc.subcore_barrier()

        def run_core(gpt, ebase):
            for b in range(NB - 1):
                idx_load(b, b, ebase)
            for b in range(NB - 1):
                idx_wait(b)
                gather_start(b)
            idx_load(NB - 1, NB - 1, ebase)

            def outer(t, carry):
                for b in range(NB):
                    g = NB * t + b
                    bn = (b + NB - 1) % NB
                    gather_wait(b)

                    @pl.when(g > 0)
                    def _():
                        scatter_wait(bn)

                    @pl.when(g + NB - 1 < gpt)
                    def _():
                        idx_wait(bn)
                        gather_start(bn)
                    scale(b)
                    scatter_start(b)

                    @pl.when(g + NB < gpt)
                    def _():
                        idx_load(g + NB, b, ebase)
                return carry

            lax.fori_loop(0, gpt // NB, outer, 0, unroll=False)
            scatter_wait(NB - 1)

        @pl.when(cid == 0)
        def _():
            run_core(gpt0, sid * (gpt0 * GB))

        @pl.when(cid == 1)
        def _():
            run_core(gpt1, NS * (gpt0 * GB) + sid * (gpt1 * GB))

        plsc.subcore_barrier()

        pltpu.sync_copy(acc.at[pl.ds(r0, ROWS_PER_TILE)],
                        out_hbm.at[cid, pl.ds(r0, ROWS_PER_TILE)])

    return spmm


_ROW_BLK = 1000


def _dense_body(a0_ref, a1_ref, prev_ref, w1_ref, w2_ref, b1_ref, b2_ref,
                out_ref):
    a = a0_ref[...] + a1_ref[...]
    x2 = prev_ref[...] * a
    out_ref[...] = (
        jnp.dot(a, w1_ref[...], preferred_element_type=jnp.float32)
        + jnp.dot(x2, w2_ref[...], preferred_element_type=jnp.float32)
        + b1_ref[...] + b2_ref[...]
    )


def _dense(a0, a1, prev, w1, w2, b1, b2):
    grid = (N // _ROW_BLK,)
    row_spec = pl.BlockSpec((_ROW_BLK, D), lambda i: (i, 0))
    full_spec = pl.BlockSpec((D, D), lambda i: (0, 0))
    bias_spec = pl.BlockSpec((1, D), lambda i: (0, 0))
    return pl.pallas_call(
        _dense_body,
        grid=grid,
        in_specs=[row_spec, row_spec, row_spec, full_spec, full_spec,
                  bias_spec, bias_spec],
        out_specs=row_spec,
        out_shape=jax.ShapeDtypeStruct((N, D), jnp.float32),
    )(a0, a1, prev, w1, w2, b1, b2)


def kernel(L_I_indices, L_I_values, prev_embeddings, W1, W2, b1, b2):
    e = L_I_values.shape[0]
    gtot = -(-e // (NS * GB))
    gpt0 = max(NB, (int(gtot * F0) // NB) * NB)
    gpt1 = -(-(gtot - gpt0) // NB) * NB
    e_pad = (gpt0 + gpt1) * NS * GB
    pad = e_pad - e

    row = L_I_indices[0]
    col = L_I_indices[1]
    if pad:
        zi = jnp.zeros((pad,), jnp.int32)
        row = jnp.concatenate([row, zi])
        col = jnp.concatenate([col, zi])
        vals = jnp.concatenate([L_I_values, jnp.zeros((pad,), jnp.float32)])
    else:
        vals = L_I_values

    zeros = jnp.zeros((NP, D), jnp.float32)
    partial = _make_spmm(gpt0, gpt1)(row, col, vals, prev_embeddings, zeros)
    return _dense(partial[0, :N], partial[1, :N], prev_embeddings, W1, W2,
                  b1, b2)


# speedup vs baseline: 1.5408x; 1.0155x over previous
import functools

import jax
import jax.numpy as jnp
from jax import lax
from jax.experimental import pallas as pl
from jax.experimental.pallas import tpu as pltpu
from jax.experimental.pallas import tpu_sc as plsc

N = 10000
D = 128
NC = 2
NS = 16
NW = NC * NS
GB = 64
NB = 4
F0 = 0.50
NP = 10240
ROWS_PER_TILE = NP // NS

_mesh = plsc.VectorSubcoreMesh(core_axis_name="c", subcore_axis_name="s")


def _make_spmm(gpt0: int, gpt1: int):
    @functools.partial(
        pl.kernel,
        mesh=_mesh,
        out_type=jax.ShapeDtypeStruct((NC, NP, D), jnp.float32),
        scratch_types=[
            [pltpu.VMEM((GB,), jnp.int32) for _ in range(NB)],
            [pltpu.VMEM((GB,), jnp.int32) for _ in range(NB)],
            [pltpu.VMEM((GB,), jnp.float32) for _ in range(NB)],
            [pltpu.VMEM((GB,), jnp.int32) for _ in range(NB)],
            [pltpu.VMEM((GB, D), jnp.float32) for _ in range(NB)],
            pltpu.VMEM_SHARED((NP, D), jnp.float32),
            [pltpu.SemaphoreType.DMA for _ in range(NB)],
            [pltpu.SemaphoreType.DMA for _ in range(NB)],
            [pltpu.SemaphoreType.DMA for _ in range(NB)],
        ],
    )
    def spmm(row_hbm, col_hbm, val_hbm, prev_hbm, zeros_hbm, out_hbm,
             rowb, colb, valb, rowsc, rbuf, acc, semg, sems, semi):
        cid = lax.axis_index("c")
        sid = lax.axis_index("s")
        r0 = sid * ROWS_PER_TILE

        def idx_load(g, b, ebase):
            off = ebase + g * GB
            pltpu.async_copy(row_hbm.at[pl.ds(off, GB)], rowb[b], semi[b])
            pltpu.async_copy(col_hbm.at[pl.ds(off, GB)], colb[b], semi[b])
            pltpu.async_copy(val_hbm.at[pl.ds(off, GB)], valb[b], semi[b])

        def idx_wait(b):
            pltpu.make_async_copy(row_hbm.at[pl.ds(0, GB)], rowb[b],
                                  semi[b]).wait()
            pltpu.make_async_copy(col_hbm.at[pl.ds(0, GB)], colb[b],
                                  semi[b]).wait()
            pltpu.make_async_copy(val_hbm.at[pl.ds(0, GB)], valb[b],
                                  semi[b]).wait()

        def gather_start(b):
            pltpu.async_copy(prev_hbm.at[colb[b]], rbuf[b], semg[b])

        def gather_wait(b):
            pltpu.make_async_copy(prev_hbm.at[colb[b]], rbuf[b],
                                  semg[b]).wait()

        def scatter_start(b):
            for j in range(GB // 16):
                rowsc[b][pl.ds(j * 16, 16)] = rowb[b][pl.ds(j * 16, 16)]
            pltpu.async_copy(rbuf[b], acc.at[rowsc[b]], sems[b], add=True)

        def scatter_wait(b):
            pltpu.make_async_copy(rbuf[b], acc.at[rowsc[b]], sems[b]).wait()

        def scale(b):
            def scale_body(k, c2):
                vv = valb[b][pl.ds(k * 16, 16)]
                for l in range(16):
                    sv = jnp.full((16,), vv[l], jnp.float32)
                    e = k * 16 + l
                    for j in range(D // 16):
                        rbuf[b][e, pl.ds(j * 16, 16)] = (
                            rbuf[b][e, pl.ds(j * 16, 16)] * sv)
                return c2

            lax.fori_loop(0, GB // 16, scale_body, 0, unroll=False)

        pltpu.sync_copy(zeros_hbm.at[pl.ds(r0, ROWS_PER_TILE)],
                        acc.at[pl.ds(r0, ROWS_PER_TILE)])
        plsc.subcore_barrier()

        def run_core(gpt, ebase):
            for b in range(NB - 1):
                idx_load(b, b, ebase)
            for b in range(NB - 1):
                idx_wait(b)
                gather_start(b)
            idx_load(NB - 1, NB - 1, ebase)

            def outer(t, carry):
                for b in range(NB):
                    g = NB * t + b
                    bn = (b + NB - 1) % NB
                    gather_wait(b)

                    @pl.when(g > 0)
                    def _():
                        scatter_wait(bn)

                    @pl.when(g + NB - 1 < gpt)
                    def _():
                        idx_wait(bn)
                        gather_start(bn)
                    scale(b)
                    scatter_start(b)

                    @pl.when(g + NB < gpt)
                    def _():
                        idx_load(g + NB, b, ebase)
                return carry

            lax.fori_loop(0, gpt // NB, outer, 0, unroll=False)
            scatter_wait(NB - 1)

        @pl.when(cid == 0)
        def _():
            run_core(gpt0, sid * (gpt0 * GB))

        @pl.when(cid == 1)
        def _():
            run_core(gpt1, NS * (gpt0 * GB) + sid * (gpt1 * GB))

        plsc.subcore_barrier()

        pltpu.sync_copy(acc.at[pl.ds(r0, ROWS_PER_TILE)],
                        out_hbm.at[cid, pl.ds(r0, ROWS_PER_TILE)])

    return spmm


_ROW_BLK = 1000


def _dense_body(a0_ref, a1_ref, prev_ref, w1_ref, w2_ref, b1_ref, b2_ref,
                out_ref):
    a = a0_ref[...] + a1_ref[...]
    x2 = prev_ref[...] * a
    out_ref[...] = (
        jnp.dot(a, w1_ref[...], preferred_element_type=jnp.float32)
        + jnp.dot(x2, w2_ref[...], preferred_element_type=jnp.float32)
        + b1_ref[...] + b2_ref[...]
    )


def _dense(a0, a1, prev, w1, w2, b1, b2):
    grid = (N // _ROW_BLK,)
    row_spec = pl.BlockSpec((_ROW_BLK, D), lambda i: (i, 0))
    full_spec = pl.BlockSpec((D, D), lambda i: (0, 0))
    bias_spec = pl.BlockSpec((1, D), lambda i: (0, 0))
    return pl.pallas_call(
        _dense_body,
        grid=grid,
        in_specs=[row_spec, row_spec, row_spec, full_spec, full_spec,
                  bias_spec, bias_spec],
        out_specs=row_spec,
        out_shape=jax.ShapeDtypeStruct((N, D), jnp.float32),
    )(a0, a1, prev, w1, w2, b1, b2)


def kernel(L_I_indices, L_I_values, prev_embeddings, W1, W2, b1, b2):
    e = L_I_values.shape[0]
    gtot = -(-e // (NS * GB))
    gpt0 = max(NB, (int(gtot * F0) // NB) * NB)
    gpt1 = -(-(gtot - gpt0) // NB) * NB
    e_pad = (gpt0 + gpt1) * NS * GB
    pad = e_pad - e

    row = L_I_indices[0]
    col = L_I_indices[1]
    if pad:
        zi = jnp.zeros((pad,), jnp.int32)
        row = jnp.concatenate([row, zi])
        col = jnp.concatenate([col, zi])
        vals = jnp.concatenate([L_I_values, jnp.zeros((pad,), jnp.float32)])
    else:
        vals = L_I_values

    zeros = jnp.zeros((NP, D), jnp.float32)
    partial = _make_spmm(gpt0, gpt1)(row, col, vals, prev_embeddings, zeros)
    return _dense(partial[0, :N], partial[1, :N], prev_embeddings, W1, W2,
                  b1, b2)


# speedup vs baseline: 1.5891x; 1.0313x over previous
import functools

import jax
import jax.numpy as jnp
from jax import lax
from jax.experimental import pallas as pl
from jax.experimental.pallas import tpu as pltpu
from jax.experimental.pallas import tpu_sc as plsc

N = 10000
D = 128
NC = 2
NS = 16
NW = NC * NS
GB = 64
NB = 4
F0 = 0.54
NP = 10240
ROWS_PER_TILE = NP // NS

_mesh = plsc.VectorSubcoreMesh(core_axis_name="c", subcore_axis_name="s")


def _make_spmm(gpt0: int, gpt1: int):
    @functools.partial(
        pl.kernel,
        mesh=_mesh,
        out_type=jax.ShapeDtypeStruct((NC, NP, D), jnp.float32),
        scratch_types=[
            [pltpu.VMEM((GB,), jnp.int32) for _ in range(NB)],
            [pltpu.VMEM((GB,), jnp.int32) for _ in range(NB)],
            [pltpu.VMEM((GB,), jnp.float32) for _ in range(NB)],
            [pltpu.VMEM((GB,), jnp.int32) for _ in range(NB)],
            [pltpu.VMEM((GB, D), jnp.float32) for _ in range(NB)],
            pltpu.VMEM_SHARED((NP, D), jnp.float32),
            [pltpu.SemaphoreType.DMA for _ in range(NB)],
            [pltpu.SemaphoreType.DMA for _ in range(NB)],
            [pltpu.SemaphoreType.DMA for _ in range(NB)],
        ],
    )
    def spmm(row_hbm, col_hbm, val_hbm, prev_hbm, zeros_hbm, out_hbm,
             rowb, colb, valb, rowsc, rbuf, acc, semg, sems, semi):
        cid = lax.axis_index("c")
        sid = lax.axis_index("s")
        r0 = sid * ROWS_PER_TILE

        def idx_load(g, b, ebase):
            off = ebase + g * GB
            pltpu.async_copy(row_hbm.at[pl.ds(off, GB)], rowb[b], semi[b])
            pltpu.async_copy(col_hbm.at[pl.ds(off, GB)], colb[b], semi[b])
            pltpu.async_copy(val_hbm.at[pl.ds(off, GB)], valb[b], semi[b])

        def idx_wait(b):
            pltpu.make_async_copy(row_hbm.at[pl.ds(0, GB)], rowb[b],
                                  semi[b]).wait()
            pltpu.make_async_copy(col_hbm.at[pl.ds(0, GB)], colb[b],
                                  semi[b]).wait()
            pltpu.make_async_copy(val_hbm.at[pl.ds(0, GB)], valb[b],
                                  semi[b]).wait()

        def gather_start(b):
            pltpu.async_copy(prev_hbm.at[colb[b]], rbuf[b], semg[b])

        def gather_wait(b):
            pltpu.make_async_copy(prev_hbm.at[colb[b]], rbuf[b],
                                  semg[b]).wait()

        def scatter_start(b):
            for j in range(GB // 16):
                rowsc[b][pl.ds(j * 16, 16)] = rowb[b][pl.ds(j * 16, 16)]
            pltpu.async_copy(rbuf[b], acc.at[rowsc[b]], sems[b], add=True)

        def scatter_wait(b):
            pltpu.make_async_copy(rbuf[b], acc.at[rowsc[b]], sems[b]).wait()

        def scale(b):
            def scale_body(k, c2):
                vv = valb[b][pl.ds(k * 16, 16)]
                for l in range(16):
                    sv = jnp.full((16,), vv[l], jnp.float32)
                    e = k * 16 + l
                    for j in range(D // 16):
                        rbuf[b][e, pl.ds(j * 16, 16)] = (
                            rbuf[b][e, pl.ds(j * 16, 16)] * sv)
                return c2

            lax.fori_loop(0, GB // 16, scale_body, 0, unroll=False)

        pltpu.sync_copy(zeros_hbm.at[pl.ds(r0, ROWS_PER_TILE)],
                        acc.at[pl.ds(r0, ROWS_PER_TILE)])
        plsc.subcore_barrier()

        def run_core(gpt, ebase):
            for b in range(NB - 1):
                idx_load(b, b, ebase)
            for b in range(NB - 1):
                idx_wait(b)
                gather_start(b)
            idx_load(NB - 1, NB - 1, ebase)

            def outer(t, carry):
                for b in range(NB):
                    g = NB * t + b
                    bn = (b + NB - 1) % NB
                    gather_wait(b)

                    @pl.when(g > 0)
                    def _():
                        scatter_wait(bn)

                    @pl.when(g + NB - 1 < gpt)
                    def _():
                        idx_wait(bn)
                        gather_start(bn)
                    scale(b)
                    scatter_start(b)

                    @pl.when(g + NB < gpt)
                    def _():
                        idx_load(g + NB, b, ebase)
                return carry

            lax.fori_loop(0, gpt // NB, outer, 0, unroll=False)
            scatter_wait(NB - 1)

        @pl.when(cid == 0)
        def _():
            run_core(gpt0, sid * (gpt0 * GB))

        @pl.when(cid == 1)
        def _():
            run_core(gpt1, NS * (gpt0 * GB) + sid * (gpt1 * GB))

        plsc.subcore_barrier()

        pltpu.sync_copy(acc.at[pl.ds(r0, ROWS_PER_TILE)],
                        out_hbm.at[cid, pl.ds(r0, ROWS_PER_TILE)])

    return spmm


_ROW_BLK = 1000


def _dense_body(a0_ref, a1_ref, prev_ref, w1_ref, w2_ref, b1_ref, b2_ref,
                out_ref):
    a = a0_ref[...] + a1_ref[...]
    x2 = prev_ref[...] * a
    out_ref[...] = (
        jnp.dot(a, w1_ref[...], preferred_element_type=jnp.float32)
        + jnp.dot(x2, w2_ref[...], preferred_element_type=jnp.float32)
        + b1_ref[...] + b2_ref[...]
    )


def _dense(a0, a1, prev, w1, w2, b1, b2):
    grid = (N // _ROW_BLK,)
    row_spec = pl.BlockSpec((_ROW_BLK, D), lambda i: (i, 0))
    full_spec = pl.BlockSpec((D, D), lambda i: (0, 0))
    bias_spec = pl.BlockSpec((1, D), lambda i: (0, 0))
    return pl.pallas_call(
        _dense_body,
        grid=grid,
        in_specs=[row_spec, row_spec, row_spec, full_spec, full_spec,
                  bias_spec, bias_spec],
        out_specs=row_spec,
        out_shape=jax.ShapeDtypeStruct((N, D), jnp.float32),
    )(a0, a1, prev, w1, w2, b1, b2)


def kernel(L_I_indices, L_I_values, prev_embeddings, W1, W2, b1, b2):
    e = L_I_values.shape[0]
    gtot = -(-e // (NS * GB))
    gpt0 = max(NB, (int(gtot * F0) // NB) * NB)
    gpt1 = -(-(gtot - gpt0) // NB) * NB
    e_pad = (gpt0 + gpt1) * NS * GB
    pad = e_pad - e

    row = L_I_indices[0]
    col = L_I_indices[1]
    if pad:
        zi = jnp.zeros((pad,), jnp.int32)
        row = jnp.concatenate([row, zi])
        col = jnp.concatenate([col, zi])
        vals = jnp.concatenate([L_I_values, jnp.zeros((pad,), jnp.float32)])
    else:
        vals = L_I_values

    zeros = jnp.zeros((NP, D), jnp.float32)
    partial = _make_spmm(gpt0, gpt1)(row, col, vals, prev_embeddings, zeros)
    return _dense(partial[0, :N], partial[1, :N], prev_embeddings, W1, W2,
                  b1, b2)


# speedup vs baseline: 1.6374x; 1.0304x over previous
import functools

import jax
import jax.numpy as jnp
from jax import lax
from jax.experimental import pallas as pl
from jax.experimental.pallas import tpu as pltpu
from jax.experimental.pallas import tpu_sc as plsc

N = 10000
D = 128
NC = 2
NS = 16
NW = NC * NS
GB = 64
NB = 4
F0 = 0.60
NP = 10240
ROWS_PER_TILE = NP // NS

_mesh = plsc.VectorSubcoreMesh(core_axis_name="c", subcore_axis_name="s")


def _make_spmm(gpt0: int, gpt1: int):
    @functools.partial(
        pl.kernel,
        mesh=_mesh,
        out_type=jax.ShapeDtypeStruct((NC, NP, D), jnp.float32),
        scratch_types=[
            [pltpu.VMEM((GB,), jnp.int32) for _ in range(NB)],
            [pltpu.VMEM((GB,), jnp.int32) for _ in range(NB)],
            [pltpu.VMEM((GB,), jnp.float32) for _ in range(NB)],
            [pltpu.VMEM((GB,), jnp.int32) for _ in range(NB)],
            [pltpu.VMEM((GB, D), jnp.float32) for _ in range(NB)],
            pltpu.VMEM_SHARED((NP, D), jnp.float32),
            [pltpu.SemaphoreType.DMA for _ in range(NB)],
            [pltpu.SemaphoreType.DMA for _ in range(NB)],
            [pltpu.SemaphoreType.DMA for _ in range(NB)],
        ],
    )
    def spmm(row_hbm, col_hbm, val_hbm, prev_hbm, zeros_hbm, out_hbm,
             rowb, colb, valb, rowsc, rbuf, acc, semg, sems, semi):
        cid = lax.axis_index("c")
        sid = lax.axis_index("s")
        r0 = sid * ROWS_PER_TILE

        def idx_load(g, b, ebase):
            off = ebase + g * GB
            pltpu.async_copy(row_hbm.at[pl.ds(off, GB)], rowb[b], semi[b])
            pltpu.async_copy(col_hbm.at[pl.ds(off, GB)], colb[b], semi[b])
            pltpu.async_copy(val_hbm.at[pl.ds(off, GB)], valb[b], semi[b])

        def idx_wait(b):
            pltpu.make_async_copy(row_hbm.at[pl.ds(0, GB)], rowb[b],
                                  semi[b]).wait()
            pltpu.make_async_copy(col_hbm.at[pl.ds(0, GB)], colb[b],
                                  semi[b]).wait()
            pltpu.make_async_copy(val_hbm.at[pl.ds(0, GB)], valb[b],
                                  semi[b]).wait()

        def gather_start(b):
            pltpu.async_copy(prev_hbm.at[colb[b]], rbuf[b], semg[b])

        def gather_wait(b):
            pltpu.make_async_copy(prev_hbm.at[colb[b]], rbuf[b],
                                  semg[b]).wait()

        def scatter_start(b):
            for j in range(GB // 16):
                rowsc[b][pl.ds(j * 16, 16)] = rowb[b][pl.ds(j * 16, 16)]
            pltpu.async_copy(rbuf[b], acc.at[rowsc[b]], sems[b], add=True)

        def scatter_wait(b):
            pltpu.make_async_copy(rbuf[b], acc.at[rowsc[b]], sems[b]).wait()

        def scale(b):
            def scale_body(k, c2):
                vv = valb[b][pl.ds(k * 16, 16)]
                for l in range(16):
                    sv = jnp.full((16,), vv[l], jnp.float32)
                    e = k * 16 + l
                    for j in range(D // 16):
                        rbuf[b][e, pl.ds(j * 16, 16)] = (
                            rbuf[b][e, pl.ds(j * 16, 16)] * sv)
                return c2

            lax.fori_loop(0, GB // 16, scale_body, 0, unroll=False)

        pltpu.sync_copy(zeros_hbm.at[pl.ds(r0, ROWS_PER_TILE)],
                        acc.at[pl.ds(r0, ROWS_PER_TILE)])
        plsc.subcore_barrier()

        def run_core(gpt, ebase):
            for b in range(NB - 1):
                idx_load(b, b, ebase)
            for b in range(NB - 1):
                idx_wait(b)
                gather_start(b)
            idx_load(NB - 1, NB - 1, ebase)

            def outer(t, carry):
                for b in range(NB):
                    g = NB * t + b
                    bn = (b + NB - 1) % NB
                    gather_wait(b)

                    @pl.when(g > 0)
                    def _():
                        scatter_wait(bn)

                    @pl.when(g + NB - 1 < gpt)
                    def _():
                        idx_wait(bn)
                        gather_start(bn)
                    scale(b)
                    scatter_start(b)

                    @pl.when(g + NB < gpt)
                    def _():
                        idx_load(g + NB, b, ebase)
                return carry

            lax.fori_loop(0, gpt // NB, outer, 0, unroll=False)
            scatter_wait(NB - 1)

        @pl.when(cid == 0)
        def _():
            run_core(gpt0, sid * (gpt0 * GB))

        @pl.when(cid == 1)
        def _():
            run_core(gpt1, NS * (gpt0 * GB) + sid * (gpt1 * GB))

        plsc.subcore_barrier()

        pltpu.sync_copy(acc.at[pl.ds(r0, ROWS_PER_TILE)],
                        out_hbm.at[cid, pl.ds(r0, ROWS_PER_TILE)])

    return spmm


_ROW_BLK = 1000


def _dense_body(a0_ref, a1_ref, prev_ref, w1_ref, w2_ref, b1_ref, b2_ref,
                out_ref):
    a = a0_ref[...] + a1_ref[...]
    x2 = prev_ref[...] * a
    out_ref[...] = (
        jnp.dot(a, w1_ref[...], preferred_element_type=jnp.float32)
        + jnp.dot(x2, w2_ref[...], preferred_element_type=jnp.float32)
        + b1_ref[...] + b2_ref[...]
    )


def _dense(a0, a1, prev, w1, w2, b1, b2):
    grid = (N // _ROW_BLK,)
    row_spec = pl.BlockSpec((_ROW_BLK, D), lambda i: (i, 0))
    full_spec = pl.BlockSpec((D, D), lambda i: (0, 0))
    bias_spec = pl.BlockSpec((1, D), lambda i: (0, 0))
    return pl.pallas_call(
        _dense_body,
        grid=grid,
        in_specs=[row_spec, row_spec, row_spec, full_spec, full_spec,
                  bias_spec, bias_spec],
        out_specs=row_spec,
        out_shape=jax.ShapeDtypeStruct((N, D), jnp.float32),
    )(a0, a1, prev, w1, w2, b1, b2)


def kernel(L_I_indices, L_I_values, prev_embeddings, W1, W2, b1, b2):
    e = L_I_values.shape[0]
    gtot = -(-e // (NS * GB))
    gpt0 = max(NB, (int(gtot * F0) // NB) * NB)
    gpt1 = -(-(gtot - gpt0) // NB) * NB
    e_pad = (gpt0 + gpt1) * NS * GB
    pad = e_pad - e

    row = L_I_indices[0]
    col = L_I_indices[1]
    if pad:
        zi = jnp.zeros((pad,), jnp.int32)
        row = jnp.concatenate([row, zi])
        col = jnp.concatenate([col, zi])
        vals = jnp.concatenate([L_I_values, jnp.zeros((pad,), jnp.float32)])
    else:
        vals = L_I_values

    zeros = jnp.zeros((NP, D), jnp.float32)
    partial = _make_spmm(gpt0, gpt1)(row, col, vals, prev_embeddings, zeros)
    return _dense(partial[0, :N], partial[1, :N], prev_embeddings, W1, W2,
                  b1, b2)


# speedup vs baseline: 1.7749x; 1.0840x over previous
import functools

import jax
import jax.numpy as jnp
from jax import lax
from jax.experimental import pallas as pl
from jax.experimental.pallas import tpu as pltpu
from jax.experimental.pallas import tpu_sc as plsc

N = 10000
D = 128
NC = 2
NS = 16
NW = NC * NS
GB = 64
NB = 4
F0 = 0.68
NP = 10240
ROWS_PER_TILE = NP // NS

_mesh = plsc.VectorSubcoreMesh(core_axis_name="c", subcore_axis_name="s")


def _make_spmm(gpt0: int, gpt1: int):
    @functools.partial(
        pl.kernel,
        mesh=_mesh,
        out_type=jax.ShapeDtypeStruct((NC, NP, D), jnp.float32),
        scratch_types=[
            [pltpu.VMEM((GB,), jnp.int32) for _ in range(NB)],
            [pltpu.VMEM((GB,), jnp.int32) for _ in range(NB)],
            [pltpu.VMEM((GB,), jnp.float32) for _ in range(NB)],
            [pltpu.VMEM((GB,), jnp.int32) for _ in range(NB)],
            [pltpu.VMEM((GB, D), jnp.float32) for _ in range(NB)],
            pltpu.VMEM_SHARED((NP, D), jnp.float32),
            [pltpu.SemaphoreType.DMA for _ in range(NB)],
            [pltpu.SemaphoreType.DMA for _ in range(NB)],
            [pltpu.SemaphoreType.DMA for _ in range(NB)],
        ],
    )
    def spmm(row_hbm, col_hbm, val_hbm, prev_hbm, zeros_hbm, out_hbm,
             rowb, colb, valb, rowsc, rbuf, acc, semg, sems, semi):
        cid = lax.axis_index("c")
        sid = lax.axis_index("s")
        r0 = sid * ROWS_PER_TILE

        def idx_load(g, b, ebase):
            off = ebase + g * GB
            pltpu.async_copy(row_hbm.at[pl.ds(off, GB)], rowb[b], semi[b])
            pltpu.async_copy(col_hbm.at[pl.ds(off, GB)], colb[b], semi[b])
            pltpu.async_copy(val_hbm.at[pl.ds(off, GB)], valb[b], semi[b])

        def idx_wait(b):
            pltpu.make_async_copy(row_hbm.at[pl.ds(0, GB)], rowb[b],
                                  semi[b]).wait()
            pltpu.make_async_copy(col_hbm.at[pl.ds(0, GB)], colb[b],
                                  semi[b]).wait()
            pltpu.make_async_copy(val_hbm.at[pl.ds(0, GB)], valb[b],
                                  semi[b]).wait()

        def gather_start(b):
            pltpu.async_copy(prev_hbm.at[colb[b]], rbuf[b], semg[b])

        def gather_wait(b):
            pltpu.make_async_copy(prev_hbm.at[colb[b]], rbuf[b],
                                  semg[b]).wait()

        def scatter_start(b):
            for j in range(GB // 16):
                rowsc[b][pl.ds(j * 16, 16)] = rowb[b][pl.ds(j * 16, 16)]
            pltpu.async_copy(rbuf[b], acc.at[rowsc[b]], sems[b], add=True)

        def scatter_wait(b):
            pltpu.make_async_copy(rbuf[b], acc.at[rowsc[b]], sems[b]).wait()

        def scale(b):
            def scale_body(k, c2):
                vv = valb[b][pl.ds(k * 16, 16)]
                for l in range(16):
                    sv = jnp.full((16,), vv[l], jnp.float32)
                    e = k * 16 + l
                    for j in range(D // 16):
                        rbuf[b][e, pl.ds(j * 16, 16)] = (
                            rbuf[b][e, pl.ds(j * 16, 16)] * sv)
                return c2

            lax.fori_loop(0, GB // 16, scale_body, 0, unroll=False)

        pltpu.sync_copy(zeros_hbm.at[pl.ds(r0, ROWS_PER_TILE)],
                        acc.at[pl.ds(r0, ROWS_PER_TILE)])
        plsc.subcore_barrier()

        def run_core(gpt, ebase):
            for b in range(NB - 1):
                idx_load(b, b, ebase)
            for b in range(NB - 1):
                idx_wait(b)
                gather_start(b)
            idx_load(NB - 1, NB - 1, ebase)

            def outer(t, carry):
                for b in range(NB):
                    g = NB * t + b
                    bn = (b + NB - 1) % NB
                    gather_wait(b)

                    @pl.when(g > 0)
                    def _():
                        scatter_wait(bn)

                    @pl.when(g + NB - 1 < gpt)
                    def _():
                        idx_wait(bn)
                        gather_start(bn)
                    scale(b)
                    scatter_start(b)

                    @pl.when(g + NB < gpt)
                    def _():
                        idx_load(g + NB, b, ebase)
                return carry

            lax.fori_loop(0, gpt // NB, outer, 0, unroll=False)
            scatter_wait(NB - 1)

        @pl.when(cid == 0)
        def _():
            run_core(gpt0, sid * (gpt0 * GB))

        @pl.when(cid == 1)
        def _():
            run_core(gpt1, NS * (gpt0 * GB) + sid * (gpt1 * GB))

        plsc.subcore_barrier()

        pltpu.sync_copy(acc.at[pl.ds(r0, ROWS_PER_TILE)],
                        out_hbm.at[cid, pl.ds(r0, ROWS_PER_TILE)])

    return spmm


_ROW_BLK = 1000


def _dense_body(a0_ref, a1_ref, prev_ref, w1_ref, w2_ref, b1_ref, b2_ref,
                out_ref):
    a = a0_ref[...] + a1_ref[...]
    x2 = prev_ref[...] * a
    out_ref[...] = (
        jnp.dot(a, w1_ref[...], preferred_element_type=jnp.float32)
        + jnp.dot(x2, w2_ref[...], preferred_element_type=jnp.float32)
        + b1_ref[...] + b2_ref[...]
    )


def _dense(a0, a1, prev, w1, w2, b1, b2):
    grid = (N // _ROW_BLK,)
    row_spec = pl.BlockSpec((_ROW_BLK, D), lambda i: (i, 0))
    full_spec = pl.BlockSpec((D, D), lambda i: (0, 0))
    bias_spec = pl.BlockSpec((1, D), lambda i: (0, 0))
    return pl.pallas_call(
        _dense_body,
        grid=grid,
        in_specs=[row_spec, row_spec, row_spec, full_spec, full_spec,
                  bias_spec, bias_spec],
        out_specs=row_spec,
        out_shape=jax.ShapeDtypeStruct((N, D), jnp.float32),
    )(a0, a1, prev, w1, w2, b1, b2)


def kernel(L_I_indices, L_I_values, prev_embeddings, W1, W2, b1, b2):
    e = L_I_values.shape[0]
    gtot = -(-e // (NS * GB))
    gpt0 = max(NB, (int(gtot * F0) // NB) * NB)
    gpt1 = -(-(gtot - gpt0) // NB) * NB
    e_pad = (gpt0 + gpt1) * NS * GB
    pad = e_pad - e

    row = L_I_indices[0]
    col = L_I_indices[1]
    if pad:
        zi = jnp.zeros((pad,), jnp.int32)
        row = jnp.concatenate([row, zi])
        col = jnp.concatenate([col, zi])
        vals = jnp.concatenate([L_I_values, jnp.zeros((pad,), jnp.float32)])
    else:
        vals = L_I_values

    zeros = jnp.zeros((NP, D), jnp.float32)
    partial = _make_spmm(gpt0, gpt1)(row, col, vals, prev_embeddings, zeros)
    return _dense(partial[0, :N], partial[1, :N], prev_embeddings, W1, W2,
                  b1, b2)
